# baseline (speedup 1.0000x reference)
import functools

import jax
import jax.numpy as jnp
from jax import lax
from jax.experimental import pallas as pl
from jax.experimental.pallas import tpu as pltpu
from jax.experimental.pallas import tpu_sc as plsc

NC = 2
NS = 16
NW = NC * NS
L = 16

BATCH = 16384
D = 64
B_PER_W = BATCH // NW
CHUNK = 128
NCHUNK = B_PER_W // CHUNK


def _sc_body(h_hbm, r_hbm, t_hbm, ere_hbm, eim_hbm, rre_hbm, rim_hbm,
             out_hbm, idx_h, idx_r, idx_t,
             hr_b, hi_b, tr_b, ti_b, rr_b, ri_b, out_v, sem):
    wid = lax.axis_index("s") * NC + lax.axis_index("c")
    base = wid * B_PER_W

    for c in range(NCHUNK):
        off = base + c * CHUNK
        pltpu.sync_copy(h_hbm.at[pl.ds(off, CHUNK)], idx_h.at[c])
        pltpu.sync_copy(r_hbm.at[pl.ds(off, CHUNK)], idx_r.at[c])
        pltpu.sync_copy(t_hbm.at[pl.ds(off, CHUNK)], idx_t.at[c])

    rows0 = lax.iota(jnp.int32, L)

    for c in range(NCHUNK):
        copies = [
            pltpu.async_copy(ere_hbm.at[idx_h.at[c]], hr_b, sem),
            pltpu.async_copy(eim_hbm.at[idx_h.at[c]], hi_b, sem),
            pltpu.async_copy(ere_hbm.at[idx_t.at[c]], tr_b, sem),
            pltpu.async_copy(eim_hbm.at[idx_t.at[c]], ti_b, sem),
            pltpu.async_copy(rre_hbm.at[idx_r.at[c]], rr_b, sem),
            pltpu.async_copy(rim_hbm.at[idx_r.at[c]], ri_b, sem),
        ]
        for cp in copies:
            cp.wait()

        for g in range(CHUNK // L):
            def row_step(j, out_vec):
                i = g * L + j
                acc = jnp.zeros((L,), jnp.float32)
                for s in range(D // L):
                    sl = pl.ds(s * L, L)
                    hr = hr_b[i, sl]
                    hi = hi_b[i, sl]
                    tr = tr_b[i, sl]
                    ti = ti_b[i, sl]
                    rr = rr_b[i, sl]
                    ri = ri_b[i, sl]
                    a = hr * rr - hi * ri
                    b = hi * rr + hr * ri
                    acc = acc + a * tr + b * ti
                return jnp.where(rows0 == j, jnp.sum(acc), out_vec)

            out_vec = lax.fori_loop(0, L, row_step,
                                    jnp.zeros((L,), jnp.float32))
            out_v[pl.ds(c * CHUNK + g * L, L)] = out_vec

    pltpu.sync_copy(out_v, out_hbm.at[pl.ds(base, B_PER_W)])


@jax.jit
def _complex_score(h, r, t, entity_re, entity_im, relation_re, relation_im):
    mesh = plsc.VectorSubcoreMesh(core_axis_name="c", subcore_axis_name="s")
    run = functools.partial(
        pl.kernel,
        out_type=jax.ShapeDtypeStruct((BATCH,), jnp.float32),
        mesh=mesh,
        compiler_params=pltpu.CompilerParams(needs_layout_passes=False,
                                             use_tc_tiling_on_sc=False),
        scratch_types=[
            pltpu.VMEM((NCHUNK, CHUNK), jnp.int32),
            pltpu.VMEM((NCHUNK, CHUNK), jnp.int32),
            pltpu.VMEM((NCHUNK, CHUNK), jnp.int32),
            pltpu.VMEM((CHUNK, D), jnp.float32),
            pltpu.VMEM((CHUNK, D), jnp.float32),
            pltpu.VMEM((CHUNK, D), jnp.float32),
            pltpu.VMEM((CHUNK, D), jnp.float32),
            pltpu.VMEM((CHUNK, D), jnp.float32),
            pltpu.VMEM((CHUNK, D), jnp.float32),
            pltpu.VMEM((B_PER_W,), jnp.float32),
            pltpu.SemaphoreType.DMA,
        ],
    )(_sc_body)
    return run(h, r, t, entity_re, entity_im, relation_re, relation_im)


def kernel(h, r, t, entity_re, entity_im, relation_re, relation_im):
    return _complex_score(h.astype(jnp.int32), r.astype(jnp.int32),
                          t.astype(jnp.int32), entity_re, entity_im,
                          relation_re, relation_im)


# speedup vs baseline: 1.5429x; 1.5429x over previous
import functools

import jax
import jax.numpy as jnp
from jax import lax
from jax.experimental import pallas as pl
from jax.experimental.pallas import tpu as pltpu
from jax.experimental.pallas import tpu_sc as plsc

NC = 2
NS = 16
NW = NC * NS
L = 16

BATCH = 16384
D = 64
B_PER_W = BATCH // NW
CHUNK = 128
NCHUNK = B_PER_W // CHUNK
NGROUP = CHUNK // L


def _sc_body(h_hbm, r_hbm, t_hbm, ere_hbm, eim_hbm, rre_hbm, rim_hbm,
             out_hbm, idx_h, idx_r, idx_t,
             hr_b, hi_b, tr_b, ti_b, rr_b, ri_b, out_v, sem):
    wid = lax.axis_index("s") * NC + lax.axis_index("c")
    base = wid * B_PER_W

    rows0 = lax.iota(jnp.int32, L)

    def chunk_body(c):
        off = base + c * CHUNK
        pltpu.sync_copy(h_hbm.at[pl.ds(off, CHUNK)], idx_h)
        pltpu.sync_copy(r_hbm.at[pl.ds(off, CHUNK)], idx_r)
        pltpu.sync_copy(t_hbm.at[pl.ds(off, CHUNK)], idx_t)

        def group_dma(g, carry):
            gs = pl.ds(g * L, L)
            ihv = idx_h[gs]
            irv = idx_r[gs]
            itv = idx_t[gs]
            for j in range(L):
                m = rows0 == j
                ih = jnp.sum(jnp.where(m, ihv, 0))
                ir = jnp.sum(jnp.where(m, irv, 0))
                it = jnp.sum(jnp.where(m, itv, 0))
                dst = pl.ds(g * L + j, 1)
                pltpu.async_copy(ere_hbm.at[pl.ds(ih, 1)], hr_b.at[dst], sem)
                pltpu.async_copy(eim_hbm.at[pl.ds(ih, 1)], hi_b.at[dst], sem)
                pltpu.async_copy(ere_hbm.at[pl.ds(it, 1)], tr_b.at[dst], sem)
                pltpu.async_copy(eim_hbm.at[pl.ds(it, 1)], ti_b.at[dst], sem)
                pltpu.async_copy(rre_hbm.at[pl.ds(ir, 1)], rr_b.at[dst], sem)
                pltpu.async_copy(rim_hbm.at[pl.ds(ir, 1)], ri_b.at[dst], sem)
            return carry

        lax.fori_loop(0, NGROUP, group_dma, 0)

        pltpu.make_async_copy(ere_hbm.at[pl.ds(0, CHUNK)], hr_b, sem).wait()
        pltpu.make_async_copy(ere_hbm.at[pl.ds(0, CHUNK)], hi_b, sem).wait()
        pltpu.make_async_copy(ere_hbm.at[pl.ds(0, CHUNK)], tr_b, sem).wait()
        pltpu.make_async_copy(ere_hbm.at[pl.ds(0, CHUNK)], ti_b, sem).wait()
        pltpu.make_async_copy(ere_hbm.at[pl.ds(0, CHUNK)], rr_b, sem).wait()
        pltpu.make_async_copy(ere_hbm.at[pl.ds(0, CHUNK)], ri_b, sem).wait()

        def group_compute(g, carry):
            def row_step(j, out_vec):
                i = g * L + j
                acc = jnp.zeros((L,), jnp.float32)
                for s in range(D // L):
                    sl = pl.ds(s * L, L)
                    hr = hr_b[i, sl]
                    hi = hi_b[i, sl]
                    tr = tr_b[i, sl]
                    ti = ti_b[i, sl]
                    rr = rr_b[i, sl]
                    ri = ri_b[i, sl]
                    a = hr * rr - hi * ri
                    b = hi * rr + hr * ri
                    acc = acc + a * tr + b * ti
                return jnp.where(rows0 == j, jnp.sum(acc), out_vec)

            out_vec = lax.fori_loop(0, L, row_step,
                                    jnp.zeros((L,), jnp.float32))
            out_v[pl.ds(c * CHUNK + g * L, L)] = out_vec
            return carry

        lax.fori_loop(0, NGROUP, group_compute, 0)

    for c in range(NCHUNK):
        chunk_body(c)

    pltpu.sync_copy(out_v, out_hbm.at[pl.ds(base, B_PER_W)])


@jax.jit
def _complex_score(h, r, t, entity_re, entity_im, relation_re, relation_im):
    mesh = plsc.VectorSubcoreMesh(core_axis_name="c", subcore_axis_name="s")
    run = functools.partial(
        pl.kernel,
        out_type=jax.ShapeDtypeStruct((BATCH,), jnp.float32),
        mesh=mesh,
        compiler_params=pltpu.CompilerParams(needs_layout_passes=False),
        scratch_types=[
            pltpu.VMEM((CHUNK,), jnp.int32),
            pltpu.VMEM((CHUNK,), jnp.int32),
            pltpu.VMEM((CHUNK,), jnp.int32),
            pltpu.VMEM((CHUNK, D), jnp.float32),
            pltpu.VMEM((CHUNK, D), jnp.float32),
            pltpu.VMEM((CHUNK, D), jnp.float32),
            pltpu.VMEM((CHUNK, D), jnp.float32),
            pltpu.VMEM((CHUNK, D), jnp.float32),
            pltpu.VMEM((CHUNK, D), jnp.float32),
            pltpu.VMEM((B_PER_W,), jnp.float32),
            pltpu.SemaphoreType.DMA,
        ],
    )(_sc_body)
    return run(h, r, t, entity_re, entity_im, relation_re, relation_im)


def kernel(h, r, t, entity_re, entity_im, relation_re, relation_im):
    return _complex_score(h.astype(jnp.int32), r.astype(jnp.int32),
                          t.astype(jnp.int32), entity_re, entity_im,
                          relation_re, relation_im)
